# baseline (speedup 1.0000x reference)
import functools

import jax
import jax.numpy as jnp
from jax import lax
from jax.experimental import pallas as pl
from jax.experimental.pallas import tpu as pltpu
from jax.experimental.pallas import tpu_sc as plsc

V_TH = 1.0
_CH = 128
_RB = 512
_TCHUNK = 2


def _sc_gather(table, flat_idx):
    M = flat_idx.shape[0]
    D = table.shape[1]
    info = plsc.get_sparse_core_info()
    nw = info.num_cores * info.num_subcores
    rows_per_w = M // nw
    n_chunks = rows_per_w // _CH
    mesh = plsc.VectorSubcoreMesh(core_axis_name="c", subcore_axis_name="s")

    @functools.partial(
        pl.kernel,
        out_type=jax.ShapeDtypeStruct((M, D), jnp.float32),
        mesh=mesh,
        scratch_types=[
            pltpu.VMEM((_CH,), jnp.int32),
            pltpu.VMEM((_CH, D), jnp.float32),
            pltpu.SemaphoreType.DMA,
        ],
    )
    def gather_kernel(table_hbm, idx_hbm, out_hbm, idx_v, rows_v, sem):
        wid = lax.axis_index("s") * info.num_cores + lax.axis_index("c")
        tile_base = wid * rows_per_w

        @pl.loop(0, n_chunks)
        def _chunk(c):
            base = tile_base + c * _CH
            pltpu.sync_copy(idx_hbm.at[pl.ds(base, _CH)], idx_v)
            pltpu.async_copy(table_hbm.at[idx_v], rows_v, sem).wait()
            pltpu.sync_copy(rows_v, out_hbm.at[pl.ds(base, _CH)])

    return gather_kernel(table, flat_idx)


def _tc_compute(g4, Wr0, Wl0, b0, Wr1, Wl1, b1, G, W_ro, b_ro, feat_in):
    Tc, nslot, B, D = g4.shape
    H1 = Wr0.shape[1]
    H2 = Wr1.shape[1]
    C = W_ro.shape[1]
    nb = B // _RB
    dot = functools.partial(
        jnp.dot, preferred_element_type=jnp.float32,
        precision=jax.lax.Precision.HIGHEST)
    dot_h = dot

    def body(g_ref, wr0, wl0, b0r, wr1, wl1, b1r, g_row, wro, bror,
             feat_in_ref, feat_ref, out_ref):
        t = pl.program_id(1)
        g = g_ref[0]
        self6 = g[0:6].reshape(6 * _RB, D)
        yr = dot_h(self6, wr0[...]).reshape(6, _RB, H1)
        mh1 = (g[1] + g[2] + g[3] + g[4] + g[5]) * 0.2
        mh2 = (g[6:11] + g[11:16]) * 0.5
        yl0 = dot_h(mh1, wl0[...])
        ylm = dot_h(mh2.reshape(5 * _RB, D), wl0[...]).reshape(5, _RB, H1)
        s_a = (yr[0] + yl0 + b0r[...] >= V_TH).astype(jnp.float32)
        s_b = (yr[1:6] + ylm + b0r[...][None] >= V_TH).astype(jnp.float32)
        neigh1 = (s_b[0] + s_b[1] + s_b[2] + s_b[3] + s_b[4]) * 0.2
        out1 = dot(s_a, wr1[...]) + dot(neigh1, wl1[...]) + b1r[...]
        s1 = (out1 >= V_TH).astype(jnp.float32)
        contrib = s1 * g_row[0]

        @pl.when(t == 0)
        def _():
            feat_ref[...] = feat_in_ref[...] + contrib

        @pl.when(t > 0)
        def _():
            feat_ref[...] = feat_ref[...] + contrib

        @pl.when(t == Tc - 1)
        def _():
            out_ref[...] = dot(feat_ref[...], wro[...]) + bror[...]

    return pl.pallas_call(
        body,
        grid=(nb, Tc),
        in_specs=[
            pl.BlockSpec((1, nslot, _RB, D), lambda i, t: (t, 0, i, 0)),
            pl.BlockSpec((D, H1), lambda i, t: (0, 0)),
            pl.BlockSpec((D, H1), lambda i, t: (0, 0)),
            pl.BlockSpec((1, H1), lambda i, t: (0, 0)),
            pl.BlockSpec((H1, H2), lambda i, t: (0, 0)),
            pl.BlockSpec((H1, H2), lambda i, t: (0, 0)),
            pl.BlockSpec((1, H2), lambda i, t: (0, 0)),
            pl.BlockSpec((1, 1, H2), lambda i, t: (t, 0, 0)),
            pl.BlockSpec((H2, C), lambda i, t: (0, 0)),
            pl.BlockSpec((1, C), lambda i, t: (0, 0)),
            pl.BlockSpec((_RB, H2), lambda i, t: (i, 0)),
        ],
        out_specs=[
            pl.BlockSpec((_RB, H2), lambda i, t: (i, 0)),
            pl.BlockSpec((_RB, C), lambda i, t: (i, 0)),
        ],
        out_shape=[
            jax.ShapeDtypeStruct((B, H2), jnp.float32),
            jax.ShapeDtypeStruct((B, C), jnp.float32),
        ],
        compiler_params=pltpu.CompilerParams(
            dimension_semantics=("arbitrary", "arbitrary")),
    )(g4, Wr0, Wl0, b0, Wr1, Wl1, b1, G, W_ro, b_ro, feat_in)


def _readout_weights(delay_w, dw_kernel, T, groups):
    H2, K = dw_kernel.shape
    gsize = H2 // groups
    w = jax.nn.softmax(delay_w, axis=-1)
    wc = jnp.repeat(w, gsize, axis=0)
    tgrid = jnp.arange(T)
    j = jnp.arange(K)
    valid = ((tgrid[None, :] + K // 2 - j[:, None] >= 0)
             & (tgrid[None, :] + K // 2 - j[:, None] < T)).astype(jnp.float32)
    cnt = jnp.einsum("cj,jt->ct", dw_kernel, valid)
    delays = jnp.array((0, 1, 3, 5), dtype=jnp.int32)
    shift = tgrid[None, :] + delays[:, None]
    in_range = (shift < T)
    cnt_sh = jnp.where(in_range[None, :, :],
                       cnt[:, jnp.clip(shift, 0, T - 1)], 0.0)
    G0 = jnp.einsum("ck,ckt->ct", wc, cnt_sh)
    return (G0 / T).T


def kernel(x, nodes, nbr1, nbr2, Wr0, br0, Wl0, bl0, Wr1, br1, Wl1, bl1,
           delay_w, dw_kernel, W_ro, b_ro):
    T, N, D = x.shape
    B = nodes.shape[0]
    S1 = nbr1.shape[2]
    S2 = nbr2.shape[2]

    nslot = 1 + S1 + S1 * S2
    nodes_i = nodes.astype(jnp.int32)

    def chunk_idx(c, tc):
        i_nbr1 = jnp.transpose(nbr1[c:c + tc], (0, 2, 1)).astype(jnp.int32)
        i_nbr2 = jnp.transpose(nbr2[c:c + tc].reshape(tc, B, S1, S2),
                               (0, 3, 2, 1)).astype(jnp.int32)
        idx_all = jnp.concatenate(
            [jnp.broadcast_to(nodes_i[None, None, :], (tc, 1, B)),
             i_nbr1, i_nbr2.reshape(tc, S2 * S1, B)], axis=1)
        offs = ((jnp.arange(tc, dtype=jnp.int32) + c) * N)[:, None, None]
        return (idx_all + offs).reshape(-1)

    G = _readout_weights(delay_w, dw_kernel, T, groups=8)
    table = x.reshape(T * N, D)
    H2 = Wr1.shape[1]

    feat = jnp.zeros((B, H2), dtype=jnp.float32)
    out = None
    for c in range(0, T, _TCHUNK):
        gathered = _sc_gather(table, chunk_idx(c, _TCHUNK))
        g4 = gathered.reshape(_TCHUNK, nslot, B, D)
        feat, out = _tc_compute(
            g4, Wr0, Wl0, (br0 + bl0)[None, :], Wr1, Wl1,
            (br1 + bl1)[None, :], G[c:c + _TCHUNK, None, :], W_ro,
            b_ro[None, :], feat)
    return out


# speedup vs baseline: 1.2011x; 1.2011x over previous
import functools

import jax
import jax.numpy as jnp
from jax import lax
from jax.experimental import pallas as pl
from jax.experimental.pallas import tpu as pltpu
from jax.experimental.pallas import tpu_sc as plsc

V_TH = 1.0
_CH = 128
_RB = 512
_TCHUNK = 2


def _sc_gather(table, flat_idx):
    M = flat_idx.shape[0]
    D = table.shape[1]
    info = plsc.get_sparse_core_info()
    nw = info.num_cores * info.num_subcores
    rows_per_w = M // nw
    n_chunks = rows_per_w // _CH
    assert n_chunks >= 4 and n_chunks % 2 == 0
    idx2 = flat_idx.reshape(-1, _CH)
    mesh = plsc.VectorSubcoreMesh(core_axis_name="c", subcore_axis_name="s")

    @functools.partial(
        pl.kernel,
        out_type=jax.ShapeDtypeStruct((M, D), jnp.float32),
        mesh=mesh,
        scratch_types=[
            pltpu.VMEM((n_chunks, _CH), jnp.int32),
            pltpu.VMEM((_CH, D), jnp.float32),
            pltpu.VMEM((_CH, D), jnp.float32),
            pltpu.SemaphoreType.DMA,
            pltpu.SemaphoreType.DMA,
            pltpu.SemaphoreType.DMA,
            pltpu.SemaphoreType.DMA,
        ],
    )
    def gather_kernel(table_hbm, idx_hbm, out_hbm, idx_v, rows_a, rows_b,
                      sga, sgb, swa, swb):
        wid = lax.axis_index("s") * info.num_cores + lax.axis_index("c")
        tile_base = wid * rows_per_w
        pltpu.sync_copy(idx_hbm.at[pl.ds(wid * n_chunks, n_chunks)], idx_v)

        def g_start(c, buf, sem):
            pltpu.async_copy(table_hbm.at[idx_v.at[c]], buf, sem)

        def g_wait(c, buf, sem):
            pltpu.make_async_copy(table_hbm.at[idx_v.at[c]], buf, sem).wait()

        def w_start(c, buf, sem):
            pltpu.async_copy(
                buf, out_hbm.at[pl.ds(tile_base + c * _CH, _CH)], sem)

        def w_wait(c, buf, sem):
            pltpu.make_async_copy(
                buf, out_hbm.at[pl.ds(tile_base + c * _CH, _CH)], sem).wait()

        g_start(0, rows_a, sga)

        @pl.loop(0, n_chunks - 2, step=2)
        def _pair(c):
            @pl.when(c > 0)
            def _():
                w_wait(c - 1, rows_b, swb)
            g_start(c + 1, rows_b, sgb)
            g_wait(c, rows_a, sga)
            w_start(c, rows_a, swa)
            w_wait(c, rows_a, swa)
            g_start(c + 2, rows_a, sga)
            g_wait(c + 1, rows_b, sgb)
            w_start(c + 1, rows_b, swb)

        c0 = n_chunks - 2
        w_wait(c0 - 1, rows_b, swb)
        g_start(c0 + 1, rows_b, sgb)
        g_wait(c0, rows_a, sga)
        w_start(c0, rows_a, swa)
        g_wait(c0 + 1, rows_b, sgb)
        w_start(c0 + 1, rows_b, swb)
        w_wait(c0, rows_a, swa)
        w_wait(c0 + 1, rows_b, swb)

    return gather_kernel(table, idx2)


def _tc_compute(g4, Wr0, Wl0, b0, Wr1, Wl1, b1, G, W_ro, b_ro, feat_in):
    Tc, nslot, B, D = g4.shape
    H1 = Wr0.shape[1]
    H2 = Wr1.shape[1]
    C = W_ro.shape[1]
    nb = B // _RB
    dot = functools.partial(
        jnp.dot, preferred_element_type=jnp.float32,
        precision=jax.lax.Precision.HIGHEST)
    dot_h = dot

    def body(g_ref, wr0, wl0, b0r, wr1, wl1, b1r, g_row, wro, bror,
             feat_in_ref, feat_ref, out_ref):
        t = pl.program_id(1)
        g = g_ref[0]
        self6 = g[0:6].reshape(6 * _RB, D)
        yr = dot_h(self6, wr0[...]).reshape(6, _RB, H1)
        mh1 = (g[1] + g[2] + g[3] + g[4] + g[5]) * 0.2
        mh2 = (g[6:11] + g[11:16]) * 0.5
        yl0 = dot_h(mh1, wl0[...])
        ylm = dot_h(mh2.reshape(5 * _RB, D), wl0[...]).reshape(5, _RB, H1)
        s_a = (yr[0] + yl0 + b0r[...] >= V_TH).astype(jnp.float32)
        s_b = (yr[1:6] + ylm + b0r[...][None] >= V_TH).astype(jnp.float32)
        neigh1 = (s_b[0] + s_b[1] + s_b[2] + s_b[3] + s_b[4]) * 0.2
        out1 = dot(s_a, wr1[...]) + dot(neigh1, wl1[...]) + b1r[...]
        s1 = (out1 >= V_TH).astype(jnp.float32)
        contrib = s1 * g_row[0]

        @pl.when(t == 0)
        def _():
            feat_ref[...] = feat_in_ref[...] + contrib

        @pl.when(t > 0)
        def _():
            feat_ref[...] = feat_ref[...] + contrib

        @pl.when(t == Tc - 1)
        def _():
            out_ref[...] = dot(feat_ref[...], wro[...]) + bror[...]

    return pl.pallas_call(
        body,
        grid=(nb, Tc),
        in_specs=[
            pl.BlockSpec((1, nslot, _RB, D), lambda i, t: (t, 0, i, 0)),
            pl.BlockSpec((D, H1), lambda i, t: (0, 0)),
            pl.BlockSpec((D, H1), lambda i, t: (0, 0)),
            pl.BlockSpec((1, H1), lambda i, t: (0, 0)),
            pl.BlockSpec((H1, H2), lambda i, t: (0, 0)),
            pl.BlockSpec((H1, H2), lambda i, t: (0, 0)),
            pl.BlockSpec((1, H2), lambda i, t: (0, 0)),
            pl.BlockSpec((1, 1, H2), lambda i, t: (t, 0, 0)),
            pl.BlockSpec((H2, C), lambda i, t: (0, 0)),
            pl.BlockSpec((1, C), lambda i, t: (0, 0)),
            pl.BlockSpec((_RB, H2), lambda i, t: (i, 0)),
        ],
        out_specs=[
            pl.BlockSpec((_RB, H2), lambda i, t: (i, 0)),
            pl.BlockSpec((_RB, C), lambda i, t: (i, 0)),
        ],
        out_shape=[
            jax.ShapeDtypeStruct((B, H2), jnp.float32),
            jax.ShapeDtypeStruct((B, C), jnp.float32),
        ],
        compiler_params=pltpu.CompilerParams(
            dimension_semantics=("arbitrary", "arbitrary")),
    )(g4, Wr0, Wl0, b0, Wr1, Wl1, b1, G, W_ro, b_ro, feat_in)


def _readout_weights(delay_w, dw_kernel, T, groups):
    H2, K = dw_kernel.shape
    gsize = H2 // groups
    w = jax.nn.softmax(delay_w, axis=-1)
    wc = jnp.repeat(w, gsize, axis=0)
    tgrid = jnp.arange(T)
    j = jnp.arange(K)
    valid = ((tgrid[None, :] + K // 2 - j[:, None] >= 0)
             & (tgrid[None, :] + K // 2 - j[:, None] < T)).astype(jnp.float32)
    cnt = jnp.einsum("cj,jt->ct", dw_kernel, valid)
    delays = jnp.array((0, 1, 3, 5), dtype=jnp.int32)
    shift = tgrid[None, :] + delays[:, None]
    in_range = (shift < T)
    cnt_sh = jnp.where(in_range[None, :, :],
                       cnt[:, jnp.clip(shift, 0, T - 1)], 0.0)
    G0 = jnp.einsum("ck,ckt->ct", wc, cnt_sh)
    return (G0 / T).T


def kernel(x, nodes, nbr1, nbr2, Wr0, br0, Wl0, bl0, Wr1, br1, Wl1, bl1,
           delay_w, dw_kernel, W_ro, b_ro):
    T, N, D = x.shape
    B = nodes.shape[0]
    S1 = nbr1.shape[2]
    S2 = nbr2.shape[2]

    nslot = 1 + S1 + S1 * S2
    nodes_i = nodes.astype(jnp.int32)

    def chunk_idx(c, tc):
        i_nbr1 = jnp.transpose(nbr1[c:c + tc], (0, 2, 1)).astype(jnp.int32)
        i_nbr2 = jnp.transpose(nbr2[c:c + tc].reshape(tc, B, S1, S2),
                               (0, 3, 2, 1)).astype(jnp.int32)
        idx_all = jnp.concatenate(
            [jnp.broadcast_to(nodes_i[None, None, :], (tc, 1, B)),
             i_nbr1, i_nbr2.reshape(tc, S2 * S1, B)], axis=1)
        offs = ((jnp.arange(tc, dtype=jnp.int32) + c) * N)[:, None, None]
        return (idx_all + offs).reshape(-1)

    G = _readout_weights(delay_w, dw_kernel, T, groups=8)
    table = x.reshape(T * N, D)
    H2 = Wr1.shape[1]

    feat = jnp.zeros((B, H2), dtype=jnp.float32)
    out = None
    for c in range(0, T, _TCHUNK):
        gathered = _sc_gather(table, chunk_idx(c, _TCHUNK))
        g4 = gathered.reshape(_TCHUNK, nslot, B, D)
        feat, out = _tc_compute(
            g4, Wr0, Wl0, (br0 + bl0)[None, :], Wr1, Wl1,
            (br1 + bl1)[None, :], G[c:c + _TCHUNK, None, :], W_ro,
            b_ro[None, :], feat)
    return out


# speedup vs baseline: 1.2288x; 1.0230x over previous
import functools

import jax
import jax.numpy as jnp
from jax import lax
from jax.experimental import pallas as pl
from jax.experimental.pallas import tpu as pltpu
from jax.experimental.pallas import tpu_sc as plsc

V_TH = 1.0
_CH = 128
_RB = 512
_CHUNKS = (1, 2, 2, 2, 1)


def _sc_gather(table, flat_idx):
    M = flat_idx.shape[0]
    D = table.shape[1]
    info = plsc.get_sparse_core_info()
    nw = info.num_cores * info.num_subcores
    rows_per_w = M // nw
    n_chunks = rows_per_w // _CH
    assert n_chunks >= 4 and n_chunks % 2 == 0
    idx2 = flat_idx.reshape(-1, _CH)
    mesh = plsc.VectorSubcoreMesh(core_axis_name="c", subcore_axis_name="s")

    @functools.partial(
        pl.kernel,
        out_type=jax.ShapeDtypeStruct((M, D), jnp.float32),
        mesh=mesh,
        scratch_types=[
            pltpu.VMEM((n_chunks, _CH), jnp.int32),
            pltpu.VMEM((_CH, D), jnp.float32),
            pltpu.VMEM((_CH, D), jnp.float32),
            pltpu.SemaphoreType.DMA,
            pltpu.SemaphoreType.DMA,
            pltpu.SemaphoreType.DMA,
            pltpu.SemaphoreType.DMA,
        ],
    )
    def gather_kernel(table_hbm, idx_hbm, out_hbm, idx_v, rows_a, rows_b,
                      sga, sgb, swa, swb):
        wid = lax.axis_index("s") * info.num_cores + lax.axis_index("c")
        tile_base = wid * rows_per_w
        pltpu.sync_copy(idx_hbm.at[pl.ds(wid * n_chunks, n_chunks)], idx_v)

        def g_start(c, buf, sem):
            pltpu.async_copy(table_hbm.at[idx_v.at[c]], buf, sem)

        def g_wait(c, buf, sem):
            pltpu.make_async_copy(table_hbm.at[idx_v.at[c]], buf, sem).wait()

        def w_start(c, buf, sem):
            pltpu.async_copy(
                buf, out_hbm.at[pl.ds(tile_base + c * _CH, _CH)], sem)

        def w_wait(c, buf, sem):
            pltpu.make_async_copy(
                buf, out_hbm.at[pl.ds(tile_base + c * _CH, _CH)], sem).wait()

        g_start(0, rows_a, sga)

        @pl.loop(0, n_chunks - 2, step=2)
        def _pair(c):
            @pl.when(c > 0)
            def _():
                w_wait(c - 1, rows_b, swb)
            g_start(c + 1, rows_b, sgb)
            g_wait(c, rows_a, sga)
            w_start(c, rows_a, swa)
            w_wait(c, rows_a, swa)
            g_start(c + 2, rows_a, sga)
            g_wait(c + 1, rows_b, sgb)
            w_start(c + 1, rows_b, swb)

        c0 = n_chunks - 2
        w_wait(c0 - 1, rows_b, swb)
        g_start(c0 + 1, rows_b, sgb)
        g_wait(c0, rows_a, sga)
        w_start(c0, rows_a, swa)
        g_wait(c0 + 1, rows_b, sgb)
        w_start(c0 + 1, rows_b, swb)
        w_wait(c0, rows_a, swa)
        w_wait(c0 + 1, rows_b, swb)

    return gather_kernel(table, idx2)


def _tc_compute(g4, Wr0, Wl0, b0, Wr1, Wl1, b1, G, W_ro, b_ro, feat_in):
    Tc, nslot, B, D = g4.shape
    H1 = Wr0.shape[1]
    H2 = Wr1.shape[1]
    C = W_ro.shape[1]
    nb = B // _RB
    dot = functools.partial(
        jnp.dot, preferred_element_type=jnp.float32,
        precision=jax.lax.Precision.HIGHEST)
    dot_h = dot

    def body(g_ref, wr0, wl0, b0r, wr1, wl1, b1r, g_row, wro, bror,
             feat_in_ref, feat_ref, out_ref):
        t = pl.program_id(1)
        g = g_ref[0]
        self6 = g[0:6].reshape(6 * _RB, D)
        yr = dot_h(self6, wr0[...]).reshape(6, _RB, H1)
        mh1 = (g[1] + g[2] + g[3] + g[4] + g[5]) * 0.2
        mh2 = (g[6:11] + g[11:16]) * 0.5
        yl0 = dot_h(mh1, wl0[...])
        ylm = dot_h(mh2.reshape(5 * _RB, D), wl0[...]).reshape(5, _RB, H1)
        s_a = (yr[0] + yl0 + b0r[...] >= V_TH).astype(jnp.float32)
        s_b = (yr[1:6] + ylm + b0r[...][None] >= V_TH).astype(jnp.float32)
        neigh1 = (s_b[0] + s_b[1] + s_b[2] + s_b[3] + s_b[4]) * 0.2
        out1 = dot(s_a, wr1[...]) + dot(neigh1, wl1[...]) + b1r[...]
        s1 = (out1 >= V_TH).astype(jnp.float32)
        contrib = s1 * g_row[0]

        @pl.when(t == 0)
        def _():
            feat_ref[...] = feat_in_ref[...] + contrib

        @pl.when(t > 0)
        def _():
            feat_ref[...] = feat_ref[...] + contrib

        @pl.when(t == Tc - 1)
        def _():
            out_ref[...] = dot(feat_ref[...], wro[...]) + bror[...]

    return pl.pallas_call(
        body,
        grid=(nb, Tc),
        in_specs=[
            pl.BlockSpec((1, nslot, _RB, D), lambda i, t: (t, 0, i, 0)),
            pl.BlockSpec((D, H1), lambda i, t: (0, 0)),
            pl.BlockSpec((D, H1), lambda i, t: (0, 0)),
            pl.BlockSpec((1, H1), lambda i, t: (0, 0)),
            pl.BlockSpec((H1, H2), lambda i, t: (0, 0)),
            pl.BlockSpec((H1, H2), lambda i, t: (0, 0)),
            pl.BlockSpec((1, H2), lambda i, t: (0, 0)),
            pl.BlockSpec((1, 1, H2), lambda i, t: (t, 0, 0)),
            pl.BlockSpec((H2, C), lambda i, t: (0, 0)),
            pl.BlockSpec((1, C), lambda i, t: (0, 0)),
            pl.BlockSpec((_RB, H2), lambda i, t: (i, 0)),
        ],
        out_specs=[
            pl.BlockSpec((_RB, H2), lambda i, t: (i, 0)),
            pl.BlockSpec((_RB, C), lambda i, t: (i, 0)),
        ],
        out_shape=[
            jax.ShapeDtypeStruct((B, H2), jnp.float32),
            jax.ShapeDtypeStruct((B, C), jnp.float32),
        ],
        compiler_params=pltpu.CompilerParams(
            dimension_semantics=("arbitrary", "arbitrary")),
    )(g4, Wr0, Wl0, b0, Wr1, Wl1, b1, G, W_ro, b_ro, feat_in)


def _readout_weights(delay_w, dw_kernel, T, groups):
    H2, K = dw_kernel.shape
    gsize = H2 // groups
    w = jax.nn.softmax(delay_w, axis=-1)
    wc = jnp.repeat(w, gsize, axis=0)
    tgrid = jnp.arange(T)
    j = jnp.arange(K)
    valid = ((tgrid[None, :] + K // 2 - j[:, None] >= 0)
             & (tgrid[None, :] + K // 2 - j[:, None] < T)).astype(jnp.float32)
    cnt = jnp.einsum("cj,jt->ct", dw_kernel, valid)
    delays = jnp.array((0, 1, 3, 5), dtype=jnp.int32)
    shift = tgrid[None, :] + delays[:, None]
    in_range = (shift < T)
    cnt_sh = jnp.where(in_range[None, :, :],
                       cnt[:, jnp.clip(shift, 0, T - 1)], 0.0)
    G0 = jnp.einsum("ck,ckt->ct", wc, cnt_sh)
    return (G0 / T).T


def kernel(x, nodes, nbr1, nbr2, Wr0, br0, Wl0, bl0, Wr1, br1, Wl1, bl1,
           delay_w, dw_kernel, W_ro, b_ro):
    T, N, D = x.shape
    B = nodes.shape[0]
    S1 = nbr1.shape[2]
    S2 = nbr2.shape[2]

    nslot = 1 + S1 + S1 * S2
    nodes_i = nodes.astype(jnp.int32)

    def chunk_idx(c, tc):
        i_nbr1 = jnp.transpose(nbr1[c:c + tc], (0, 2, 1)).astype(jnp.int32)
        i_nbr2 = jnp.transpose(nbr2[c:c + tc].reshape(tc, B, S1, S2),
                               (0, 3, 2, 1)).astype(jnp.int32)
        idx_all = jnp.concatenate(
            [jnp.broadcast_to(nodes_i[None, None, :], (tc, 1, B)),
             i_nbr1, i_nbr2.reshape(tc, S2 * S1, B)], axis=1)
        offs = ((jnp.arange(tc, dtype=jnp.int32) + c) * N)[:, None, None]
        return (idx_all + offs).reshape(-1)

    G = _readout_weights(delay_w, dw_kernel, T, groups=8)
    table = x.reshape(T * N, D)
    H2 = Wr1.shape[1]

    feat = jnp.zeros((B, H2), dtype=jnp.float32)
    out = None
    c = 0
    for tc in _CHUNKS:
        gathered = _sc_gather(table, chunk_idx(c, tc))
        g4 = gathered.reshape(tc, nslot, B, D)
        feat, out = _tc_compute(
            g4, Wr0, Wl0, (br0 + bl0)[None, :], Wr1, Wl1,
            (br1 + bl1)[None, :], G[c:c + tc, None, :], W_ro,
            b_ro[None, :], feat)
        c += tc
    return out


# speedup vs baseline: 1.3824x; 1.1250x over previous
import functools

import jax
import jax.numpy as jnp
from jax import lax
from jax.experimental import pallas as pl
from jax.experimental.pallas import tpu as pltpu
from jax.experimental.pallas import tpu_sc as plsc

V_TH = 1.0
_CH = 128
_RB = 512
_CHUNKS = (1, 2, 2, 2, 1)


def _sc_gather(table, flat_idx):
    M = flat_idx.shape[0]
    D = table.shape[1]
    info = plsc.get_sparse_core_info()
    nw = info.num_cores * info.num_subcores
    rows_per_w = M // nw
    n_chunks = rows_per_w // _CH
    assert n_chunks >= 4 and n_chunks % 2 == 0
    idx2 = flat_idx.reshape(-1, _CH)
    mesh = plsc.VectorSubcoreMesh(core_axis_name="c", subcore_axis_name="s")

    @functools.partial(
        pl.kernel,
        out_type=jax.ShapeDtypeStruct((M, D), jnp.float32),
        mesh=mesh,
        scratch_types=[
            pltpu.VMEM((n_chunks, _CH), jnp.int32),
            pltpu.VMEM((_CH, D), jnp.float32),
            pltpu.VMEM((_CH, D), jnp.float32),
            pltpu.SemaphoreType.DMA,
            pltpu.SemaphoreType.DMA,
            pltpu.SemaphoreType.DMA,
            pltpu.SemaphoreType.DMA,
        ],
    )
    def gather_kernel(table_hbm, idx_hbm, out_hbm, idx_v, rows_a, rows_b,
                      sga, sgb, swa, swb):
        wid = lax.axis_index("s") * info.num_cores + lax.axis_index("c")
        tile_base = wid * rows_per_w
        pltpu.sync_copy(idx_hbm.at[pl.ds(wid * n_chunks, n_chunks)], idx_v)

        def g_start(c, buf, sem):
            pltpu.async_copy(table_hbm.at[idx_v.at[c]], buf, sem)

        def g_wait(c, buf, sem):
            pltpu.make_async_copy(table_hbm.at[idx_v.at[c]], buf, sem).wait()

        def w_start(c, buf, sem):
            pltpu.async_copy(
                buf, out_hbm.at[pl.ds(tile_base + c * _CH, _CH)], sem)

        def w_wait(c, buf, sem):
            pltpu.make_async_copy(
                buf, out_hbm.at[pl.ds(tile_base + c * _CH, _CH)], sem).wait()

        g_start(0, rows_a, sga)

        @pl.loop(0, n_chunks - 2, step=2)
        def _pair(c):
            @pl.when(c > 0)
            def _():
                w_wait(c - 1, rows_b, swb)
            g_start(c + 1, rows_b, sgb)
            g_wait(c, rows_a, sga)
            w_start(c, rows_a, swa)
            w_wait(c, rows_a, swa)
            g_start(c + 2, rows_a, sga)
            g_wait(c + 1, rows_b, sgb)
            w_start(c + 1, rows_b, swb)

        c0 = n_chunks - 2
        w_wait(c0 - 1, rows_b, swb)
        g_start(c0 + 1, rows_b, sgb)
        g_wait(c0, rows_a, sga)
        w_start(c0, rows_a, swa)
        g_wait(c0 + 1, rows_b, sgb)
        w_start(c0 + 1, rows_b, swb)
        w_wait(c0, rows_a, swa)
        w_wait(c0 + 1, rows_b, swb)

    return gather_kernel(table, idx2)


def _tc_compute(g4, W0s, b0, Wr1, Wl1, b1, G, W_ro, b_ro, feat_in):
    Tc, nslot, B, D = g4.shape
    H1 = W0s[0].shape[1]
    H2 = Wr1.shape[1]
    C = W_ro.shape[1]
    nb = B // _RB
    dot = functools.partial(
        jnp.dot, preferred_element_type=jnp.float32,
        precision=jax.lax.Precision.HIGHEST)
    dotb = functools.partial(jnp.dot, preferred_element_type=jnp.float32)

    def dot3(a, bhi, blo):
        ahi = a.astype(jnp.bfloat16)
        alo = (a - ahi.astype(jnp.float32)).astype(jnp.bfloat16)
        return dotb(ahi, bhi) + (dotb(ahi, blo) + dotb(alo, bhi))

    def body(g_ref, wr0h, wr0l, wl0h, wl0l, b0r, wr1, wl1, b1r, g_row,
             wro, bror, feat_in_ref, feat_ref, out_ref):
        t = pl.program_id(1)
        g = g_ref[0]
        self6 = g[0:6].reshape(6 * _RB, D)
        yr = dot3(self6, wr0h[...], wr0l[...]).reshape(6, _RB, H1)
        mh1 = (g[1] + g[2] + g[3] + g[4] + g[5]) * 0.2
        mh2 = (g[6:11] + g[11:16]) * 0.5
        yl0 = dot3(mh1, wl0h[...], wl0l[...])
        ylm = dot3(mh2.reshape(5 * _RB, D), wl0h[...],
                   wl0l[...]).reshape(5, _RB, H1)
        s_a = (yr[0] + yl0 + b0r[...] >= V_TH).astype(jnp.float32)
        s_b = (yr[1:6] + ylm + b0r[...][None] >= V_TH).astype(jnp.float32)
        neigh1 = (s_b[0] + s_b[1] + s_b[2] + s_b[3] + s_b[4]) * 0.2
        out1 = dot(s_a, wr1[...]) + dot(neigh1, wl1[...]) + b1r[...]
        s1 = (out1 >= V_TH).astype(jnp.float32)
        contrib = s1 * g_row[0]

        @pl.when(t == 0)
        def _():
            feat_ref[...] = feat_in_ref[...] + contrib

        @pl.when(t > 0)
        def _():
            feat_ref[...] = feat_ref[...] + contrib

        @pl.when(t == Tc - 1)
        def _():
            out_ref[...] = dot(feat_ref[...], wro[...]) + bror[...]

    return pl.pallas_call(
        body,
        grid=(nb, Tc),
        in_specs=[
            pl.BlockSpec((1, nslot, _RB, D), lambda i, t: (t, 0, i, 0)),
            pl.BlockSpec((D, H1), lambda i, t: (0, 0)),
            pl.BlockSpec((D, H1), lambda i, t: (0, 0)),
            pl.BlockSpec((D, H1), lambda i, t: (0, 0)),
            pl.BlockSpec((D, H1), lambda i, t: (0, 0)),
            pl.BlockSpec((1, H1), lambda i, t: (0, 0)),
            pl.BlockSpec((H1, H2), lambda i, t: (0, 0)),
            pl.BlockSpec((H1, H2), lambda i, t: (0, 0)),
            pl.BlockSpec((1, H2), lambda i, t: (0, 0)),
            pl.BlockSpec((1, 1, H2), lambda i, t: (t, 0, 0)),
            pl.BlockSpec((H2, C), lambda i, t: (0, 0)),
            pl.BlockSpec((1, C), lambda i, t: (0, 0)),
            pl.BlockSpec((_RB, H2), lambda i, t: (i, 0)),
        ],
        out_specs=[
            pl.BlockSpec((_RB, H2), lambda i, t: (i, 0)),
            pl.BlockSpec((_RB, C), lambda i, t: (i, 0)),
        ],
        out_shape=[
            jax.ShapeDtypeStruct((B, H2), jnp.float32),
            jax.ShapeDtypeStruct((B, C), jnp.float32),
        ],
        compiler_params=pltpu.CompilerParams(
            dimension_semantics=("arbitrary", "arbitrary")),
    )(g4, W0s[0], W0s[1], W0s[2], W0s[3], b0, Wr1, Wl1, b1, G, W_ro,
      b_ro, feat_in)


def _readout_weights(delay_w, dw_kernel, T, groups):
    H2, K = dw_kernel.shape
    gsize = H2 // groups
    w = jax.nn.softmax(delay_w, axis=-1)
    wc = jnp.repeat(w, gsize, axis=0)
    tgrid = jnp.arange(T)
    j = jnp.arange(K)
    valid = ((tgrid[None, :] + K // 2 - j[:, None] >= 0)
             & (tgrid[None, :] + K // 2 - j[:, None] < T)).astype(jnp.float32)
    cnt = jnp.einsum("cj,jt->ct", dw_kernel, valid)
    delays = jnp.array((0, 1, 3, 5), dtype=jnp.int32)
    shift = tgrid[None, :] + delays[:, None]
    in_range = (shift < T)
    cnt_sh = jnp.where(in_range[None, :, :],
                       cnt[:, jnp.clip(shift, 0, T - 1)], 0.0)
    G0 = jnp.einsum("ck,ckt->ct", wc, cnt_sh)
    return (G0 / T).T


def kernel(x, nodes, nbr1, nbr2, Wr0, br0, Wl0, bl0, Wr1, br1, Wl1, bl1,
           delay_w, dw_kernel, W_ro, b_ro):
    T, N, D = x.shape
    B = nodes.shape[0]
    S1 = nbr1.shape[2]
    S2 = nbr2.shape[2]

    nslot = 1 + S1 + S1 * S2
    nodes_i = nodes.astype(jnp.int32)

    def chunk_idx(c, tc):
        i_nbr1 = jnp.transpose(nbr1[c:c + tc], (0, 2, 1)).astype(jnp.int32)
        i_nbr2 = jnp.transpose(nbr2[c:c + tc].reshape(tc, B, S1, S2),
                               (0, 3, 2, 1)).astype(jnp.int32)
        idx_all = jnp.concatenate(
            [jnp.broadcast_to(nodes_i[None, None, :], (tc, 1, B)),
             i_nbr1, i_nbr2.reshape(tc, S2 * S1, B)], axis=1)
        offs = ((jnp.arange(tc, dtype=jnp.int32) + c) * N)[:, None, None]
        return (idx_all + offs).reshape(-1)

    G = _readout_weights(delay_w, dw_kernel, T, groups=8)
    table = x.reshape(T * N, D)
    H2 = Wr1.shape[1]

    def bf16_split(w):
        hi = w.astype(jnp.bfloat16)
        return hi, (w - hi.astype(jnp.float32)).astype(jnp.bfloat16)

    W0s = bf16_split(Wr0) + bf16_split(Wl0)

    feat = jnp.zeros((B, H2), dtype=jnp.float32)
    out = None
    c = 0
    for tc in _CHUNKS:
        gathered = _sc_gather(table, chunk_idx(c, tc))
        g4 = gathered.reshape(tc, nslot, B, D)
        feat, out = _tc_compute(
            g4, W0s, (br0 + bl0)[None, :], Wr1, Wl1,
            (br1 + bl1)[None, :], G[c:c + tc, None, :], W_ro,
            b_ro[None, :], feat)
        c += tc
    return out


# speedup vs baseline: 1.4090x; 1.0192x over previous
import functools

import jax
import jax.numpy as jnp
from jax import lax
from jax.experimental import pallas as pl
from jax.experimental.pallas import tpu as pltpu
from jax.experimental.pallas import tpu_sc as plsc

V_TH = 1.0
_CH = 128
_RB = 1024
_CHUNKS = (1, 2, 2, 2, 1)


def _sc_gather(table, flat_idx):
    M = flat_idx.shape[0]
    D = table.shape[1]
    info = plsc.get_sparse_core_info()
    nw = info.num_cores * info.num_subcores
    rows_per_w = M // nw
    n_chunks = rows_per_w // _CH
    assert n_chunks >= 4 and n_chunks % 2 == 0
    idx2 = flat_idx.reshape(-1, _CH)
    mesh = plsc.VectorSubcoreMesh(core_axis_name="c", subcore_axis_name="s")

    @functools.partial(
        pl.kernel,
        out_type=jax.ShapeDtypeStruct((M, D), jnp.float32),
        mesh=mesh,
        scratch_types=[
            pltpu.VMEM((n_chunks, _CH), jnp.int32),
            pltpu.VMEM((_CH, D), jnp.float32),
            pltpu.VMEM((_CH, D), jnp.float32),
            pltpu.SemaphoreType.DMA,
            pltpu.SemaphoreType.DMA,
            pltpu.SemaphoreType.DMA,
            pltpu.SemaphoreType.DMA,
        ],
    )
    def gather_kernel(table_hbm, idx_hbm, out_hbm, idx_v, rows_a, rows_b,
                      sga, sgb, swa, swb):
        wid = lax.axis_index("s") * info.num_cores + lax.axis_index("c")
        tile_base = wid * rows_per_w
        pltpu.sync_copy(idx_hbm.at[pl.ds(wid * n_chunks, n_chunks)], idx_v)

        def g_start(c, buf, sem):
            pltpu.async_copy(table_hbm.at[idx_v.at[c]], buf, sem)

        def g_wait(c, buf, sem):
            pltpu.make_async_copy(table_hbm.at[idx_v.at[c]], buf, sem).wait()

        def w_start(c, buf, sem):
            pltpu.async_copy(
                buf, out_hbm.at[pl.ds(tile_base + c * _CH, _CH)], sem)

        def w_wait(c, buf, sem):
            pltpu.make_async_copy(
                buf, out_hbm.at[pl.ds(tile_base + c * _CH, _CH)], sem).wait()

        g_start(0, rows_a, sga)

        @pl.loop(0, n_chunks - 2, step=2)
        def _pair(c):
            @pl.when(c > 0)
            def _():
                w_wait(c - 1, rows_b, swb)
            g_start(c + 1, rows_b, sgb)
            g_wait(c, rows_a, sga)
            w_start(c, rows_a, swa)
            w_wait(c, rows_a, swa)
            g_start(c + 2, rows_a, sga)
            g_wait(c + 1, rows_b, sgb)
            w_start(c + 1, rows_b, swb)

        c0 = n_chunks - 2
        w_wait(c0 - 1, rows_b, swb)
        g_start(c0 + 1, rows_b, sgb)
        g_wait(c0, rows_a, sga)
        w_start(c0, rows_a, swa)
        g_wait(c0 + 1, rows_b, sgb)
        w_start(c0 + 1, rows_b, swb)
        w_wait(c0, rows_a, swa)
        w_wait(c0 + 1, rows_b, swb)

    return gather_kernel(table, idx2)


def _tc_compute(g4, W0s, b0, Wr1, Wl1, b1, G, W_ro, b_ro, feat_in):
    Tc, nslot, B, D = g4.shape
    H1 = W0s[0].shape[1]
    H2 = Wr1.shape[1]
    C = W_ro.shape[1]
    nb = B // _RB
    dot = functools.partial(
        jnp.dot, preferred_element_type=jnp.float32,
        precision=jax.lax.Precision.HIGHEST)
    dotb = functools.partial(jnp.dot, preferred_element_type=jnp.float32)

    def dot3(a, bhi, blo):
        ahi = a.astype(jnp.bfloat16)
        alo = (a - ahi.astype(jnp.float32)).astype(jnp.bfloat16)
        return dotb(ahi, bhi) + (dotb(ahi, blo) + dotb(alo, bhi))

    def body(g_ref, wr0h, wr0l, wl0h, wl0l, b0r, wr1, wl1, b1r, g_row,
             wro, bror, feat_in_ref, feat_ref, out_ref):
        t = pl.program_id(1)
        g = g_ref[0]
        self6 = g[0:6].reshape(6 * _RB, D)
        yr = dot3(self6, wr0h[...], wr0l[...]).reshape(6, _RB, H1)
        mh1 = (g[1] + g[2] + g[3] + g[4] + g[5]) * 0.2
        mh2 = (g[6:11] + g[11:16]) * 0.5
        yl0 = dot3(mh1, wl0h[...], wl0l[...])
        ylm = dot3(mh2.reshape(5 * _RB, D), wl0h[...],
                   wl0l[...]).reshape(5, _RB, H1)
        s_a = (yr[0] + yl0 + b0r[...] >= V_TH).astype(jnp.float32)
        s_b = (yr[1:6] + ylm + b0r[...][None] >= V_TH).astype(jnp.float32)
        neigh1 = (s_b[0] + s_b[1] + s_b[2] + s_b[3] + s_b[4]) * 0.2
        out1 = dot(s_a, wr1[...]) + dot(neigh1, wl1[...]) + b1r[...]
        s1 = (out1 >= V_TH).astype(jnp.float32)
        contrib = s1 * g_row[0]

        @pl.when(t == 0)
        def _():
            feat_ref[...] = feat_in_ref[...] + contrib

        @pl.when(t > 0)
        def _():
            feat_ref[...] = feat_ref[...] + contrib

        @pl.when(t == Tc - 1)
        def _():
            out_ref[...] = dot(feat_ref[...], wro[...]) + bror[...]

    return pl.pallas_call(
        body,
        grid=(nb, Tc),
        in_specs=[
            pl.BlockSpec((1, nslot, _RB, D), lambda i, t: (t, 0, i, 0)),
            pl.BlockSpec((D, H1), lambda i, t: (0, 0)),
            pl.BlockSpec((D, H1), lambda i, t: (0, 0)),
            pl.BlockSpec((D, H1), lambda i, t: (0, 0)),
            pl.BlockSpec((D, H1), lambda i, t: (0, 0)),
            pl.BlockSpec((1, H1), lambda i, t: (0, 0)),
            pl.BlockSpec((H1, H2), lambda i, t: (0, 0)),
            pl.BlockSpec((H1, H2), lambda i, t: (0, 0)),
            pl.BlockSpec((1, H2), lambda i, t: (0, 0)),
            pl.BlockSpec((1, 1, H2), lambda i, t: (t, 0, 0)),
            pl.BlockSpec((H2, C), lambda i, t: (0, 0)),
            pl.BlockSpec((1, C), lambda i, t: (0, 0)),
            pl.BlockSpec((_RB, H2), lambda i, t: (i, 0)),
        ],
        out_specs=[
            pl.BlockSpec((_RB, H2), lambda i, t: (i, 0)),
            pl.BlockSpec((_RB, C), lambda i, t: (i, 0)),
        ],
        out_shape=[
            jax.ShapeDtypeStruct((B, H2), jnp.float32),
            jax.ShapeDtypeStruct((B, C), jnp.float32),
        ],
        compiler_params=pltpu.CompilerParams(
            dimension_semantics=("arbitrary", "arbitrary")),
    )(g4, W0s[0], W0s[1], W0s[2], W0s[3], b0, Wr1, Wl1, b1, G, W_ro,
      b_ro, feat_in)


def _readout_weights(delay_w, dw_kernel, T, groups):
    H2, K = dw_kernel.shape
    gsize = H2 // groups
    w = jax.nn.softmax(delay_w, axis=-1)
    wc = jnp.repeat(w, gsize, axis=0)
    tgrid = jnp.arange(T)
    j = jnp.arange(K)
    valid = ((tgrid[None, :] + K // 2 - j[:, None] >= 0)
             & (tgrid[None, :] + K // 2 - j[:, None] < T)).astype(jnp.float32)
    cnt = jnp.einsum("cj,jt->ct", dw_kernel, valid)
    delays = jnp.array((0, 1, 3, 5), dtype=jnp.int32)
    shift = tgrid[None, :] + delays[:, None]
    in_range = (shift < T)
    cnt_sh = jnp.where(in_range[None, :, :],
                       cnt[:, jnp.clip(shift, 0, T - 1)], 0.0)
    G0 = jnp.einsum("ck,ckt->ct", wc, cnt_sh)
    return (G0 / T).T


def kernel(x, nodes, nbr1, nbr2, Wr0, br0, Wl0, bl0, Wr1, br1, Wl1, bl1,
           delay_w, dw_kernel, W_ro, b_ro):
    T, N, D = x.shape
    B = nodes.shape[0]
    S1 = nbr1.shape[2]
    S2 = nbr2.shape[2]

    nslot = 1 + S1 + S1 * S2
    nodes_i = nodes.astype(jnp.int32)

    def chunk_idx(c, tc):
        i_nbr1 = jnp.transpose(nbr1[c:c + tc], (0, 2, 1)).astype(jnp.int32)
        i_nbr2 = jnp.transpose(nbr2[c:c + tc].reshape(tc, B, S1, S2),
                               (0, 3, 2, 1)).astype(jnp.int32)
        idx_all = jnp.concatenate(
            [jnp.broadcast_to(nodes_i[None, None, :], (tc, 1, B)),
             i_nbr1, i_nbr2.reshape(tc, S2 * S1, B)], axis=1)
        offs = ((jnp.arange(tc, dtype=jnp.int32) + c) * N)[:, None, None]
        return (idx_all + offs).reshape(-1)

    G = _readout_weights(delay_w, dw_kernel, T, groups=8)
    table = x.reshape(T * N, D)
    H2 = Wr1.shape[1]

    def bf16_split(w):
        hi = w.astype(jnp.bfloat16)
        return hi, (w - hi.astype(jnp.float32)).astype(jnp.bfloat16)

    W0s = bf16_split(Wr0) + bf16_split(Wl0)

    feat = jnp.zeros((B, H2), dtype=jnp.float32)
    out = None
    c = 0
    for tc in _CHUNKS:
        gathered = _sc_gather(table, chunk_idx(c, tc))
        g4 = gathered.reshape(tc, nslot, B, D)
        feat, out = _tc_compute(
            g4, W0s, (br0 + bl0)[None, :], Wr1, Wl1,
            (br1 + bl1)[None, :], G[c:c + tc, None, :], W_ro,
            b_ro[None, :], feat)
        c += tc
    return out
